# baseline (speedup 1.0000x reference)
import functools

import jax
import jax.numpy as jnp
from jax import lax
from jax.experimental import pallas as pl
from jax.experimental.pallas import tpu as pltpu
from jax.experimental.pallas import tpu_sc as plsc

N = 131072
D = 32
NC, NS, L = 2, 16, 16
NW = NC * NS
ROWS_PER_W = N // NW
CHUNK = 256
NCHUNK = ROWS_PER_W // CHUNK
NG = 8
SWEEPS = CHUNK // (NG * L)
NBUF = 2


def _sqrt16(x):
    i = lax.bitcast_convert_type(x, jnp.int32)
    y = lax.bitcast_convert_type(jnp.int32(0x5F3759DF) - (i >> 1),
                                 jnp.float32)
    for _ in range(3):
        y = y * (1.5 - 0.5 * x * y * y)
    return x * y


def _sc_body(emb_hbm, cen_hbm, out_hbm,
             cen_v, in_v0, in_v1, out_v0, out_v1,
             isem0, isem1, osem0, osem1):
    wid = lax.axis_index("s") * NC + lax.axis_index("c")
    base = wid * ROWS_PER_W

    in_v = (in_v0, in_v1)
    out_v = (out_v0, out_v1)
    isem = (isem0, isem1)
    osem = (osem0, osem1)

    pltpu.sync_copy(cen_hbm, cen_v)

    iota32 = lax.iota(jnp.int32, L) * D
    rows = [iota32 + g * L * D for g in range(NG)]

    for b in range(NBUF):
        pltpu.async_copy(emb_hbm.at[pl.ds((base + b * CHUNK) * D, CHUNK * D)],
                         in_v[b], isem[b])

    @pl.loop(0, NCHUNK, step=NBUF)
    def _chunk_pair(c):
        for b in range(NBUF):
            cid = c + b
            pltpu.make_async_copy(emb_hbm.at[pl.ds(0, CHUNK * D)],
                                  in_v[b], isem[b]).wait()
            for s in range(SWEEPS):
                acc0 = [jnp.zeros((L,), jnp.float32) for _ in range(NG)]
                acc1 = [jnp.zeros((L,), jnp.float32) for _ in range(NG)]
                for j in range(D):
                    c0 = plsc.load_gather(cen_v, [jnp.full((L,), j, jnp.int32)])
                    c1 = plsc.load_gather(cen_v,
                                          [jnp.full((L,), D + j, jnp.int32)])
                    for g in range(NG):
                        x = plsc.load_gather(
                            in_v[b], [rows[g] + (s * NG * L * D + j)])
                        u0 = x - c0
                        acc0[g] = acc0[g] + u0 * u0
                        u1 = x - c1
                        acc1[g] = acc1[g] + u1 * u1
                if s == 0:
                    @pl.when(cid >= NBUF)
                    def _():
                        pltpu.make_async_copy(
                            out_v[b], out_hbm.at[pl.ds(base, CHUNK)],
                            osem[b]).wait()
                for g in range(NG):
                    m = jnp.minimum(acc0[g], acc1[g]) + 1e-12
                    out_v[b][pl.ds((s * NG + g) * L, L)] = _sqrt16(m)
            pltpu.async_copy(out_v[b],
                             out_hbm.at[pl.ds(base + cid * CHUNK, CHUNK)],
                             osem[b])

            @pl.when(cid + NBUF < NCHUNK)
            def _():
                pltpu.async_copy(
                    emb_hbm.at[pl.ds((base + (cid + NBUF) * CHUNK) * D,
                                     CHUNK * D)],
                    in_v[b], isem[b])

    for b in range(NBUF):
        pltpu.make_async_copy(out_v[b], out_hbm.at[pl.ds(base, CHUNK)],
                              osem[b]).wait()


def kernel(embedded, centers):
    mesh = plsc.VectorSubcoreMesh(core_axis_name="c", subcore_axis_name="s",
                                  num_cores=NC, num_subcores=NS)
    run = pl.kernel(
        _sc_body,
        out_type=jax.ShapeDtypeStruct((N,), jnp.float32),
        mesh=mesh,
        compiler_params=pltpu.CompilerParams(needs_layout_passes=False),
        scratch_types=[
            pltpu.VMEM((2 * D,), jnp.float32),
            pltpu.VMEM((CHUNK * D,), jnp.float32),
            pltpu.VMEM((CHUNK * D,), jnp.float32),
            pltpu.VMEM((CHUNK,), jnp.float32),
            pltpu.VMEM((CHUNK,), jnp.float32),
            pltpu.SemaphoreType.DMA,
            pltpu.SemaphoreType.DMA,
            pltpu.SemaphoreType.DMA,
            pltpu.SemaphoreType.DMA,
        ],
    )
    return run(embedded.reshape(N * D), centers.reshape(2 * D))


# speedup vs baseline: 1.2121x; 1.2121x over previous
import functools

import jax
import jax.numpy as jnp
from jax import lax
from jax.experimental import pallas as pl
from jax.experimental.pallas import tpu as pltpu
from jax.experimental.pallas import tpu_sc as plsc

N = 131072
D = 32
NC, NS, L = 2, 16, 16
NW = NC * NS
ROWS_PER_W = N // NW
CHUNK = 256
NCHUNK = ROWS_PER_W // CHUNK
NG = 8
SWEEPS = CHUNK // (NG * L)
NBUF = 2


def _sqrt16(x):
    i = lax.bitcast_convert_type(x, jnp.int32)
    y = lax.bitcast_convert_type(jnp.int32(0x5F3759DF) - (i >> 1),
                                 jnp.float32)
    for _ in range(3):
        y = y * (1.5 - 0.5 * x * y * y)
    return x * y


def _sc_body(emb_hbm, cen_hbm, out_hbm,
             cen_v, in_v0, in_v1, out_v0, out_v1,
             isem0, isem1, osem0, osem1):
    wid = lax.axis_index("s") * NC + lax.axis_index("c")
    base = wid * ROWS_PER_W

    in_v = (in_v0, in_v1)
    out_v = (out_v0, out_v1)
    isem = (isem0, isem1)
    osem = (osem0, osem1)

    pltpu.sync_copy(cen_hbm, cen_v)

    iota = lax.iota(jnp.int32, L)
    iota32 = iota * D
    rots = [(iota + j) & (D - 1) for j in range(D)]
    vjs = [iota32 + rots[j] for j in range(D)]

    for b in range(NBUF):
        pltpu.async_copy(emb_hbm.at[pl.ds((base + b * CHUNK) * D, CHUNK * D)],
                         in_v[b], isem[b])

    @pl.loop(0, NCHUNK, step=NBUF)
    def _chunk_pair(c):
        for b in range(NBUF):
            cid = c + b
            pltpu.make_async_copy(emb_hbm.at[pl.ds(0, CHUNK * D)],
                                  in_v[b], isem[b]).wait()
            for s in range(SWEEPS):
                acc0 = [jnp.zeros((L,), jnp.float32) for _ in range(NG)]
                acc1 = [jnp.zeros((L,), jnp.float32) for _ in range(NG)]
                for j in range(D):
                    c0 = plsc.load_gather(cen_v, [rots[j]])
                    c1 = plsc.load_gather(cen_v, [rots[j] + D])
                    for g in range(NG):
                        x = plsc.load_gather(
                            in_v[b], [vjs[j] + ((s * NG + g) * L * D)])
                        u0 = x - c0
                        acc0[g] = acc0[g] + u0 * u0
                        u1 = x - c1
                        acc1[g] = acc1[g] + u1 * u1
                if s == 0:
                    @pl.when(cid >= NBUF)
                    def _():
                        pltpu.make_async_copy(
                            out_v[b], out_hbm.at[pl.ds(base, CHUNK)],
                            osem[b]).wait()
                for g in range(NG):
                    m = jnp.minimum(acc0[g], acc1[g]) + 1e-12
                    out_v[b][pl.ds((s * NG + g) * L, L)] = _sqrt16(m)
            pltpu.async_copy(out_v[b],
                             out_hbm.at[pl.ds(base + cid * CHUNK, CHUNK)],
                             osem[b])

            @pl.when(cid + NBUF < NCHUNK)
            def _():
                pltpu.async_copy(
                    emb_hbm.at[pl.ds((base + (cid + NBUF) * CHUNK) * D,
                                     CHUNK * D)],
                    in_v[b], isem[b])

    for b in range(NBUF):
        pltpu.make_async_copy(out_v[b], out_hbm.at[pl.ds(base, CHUNK)],
                              osem[b]).wait()


def kernel(embedded, centers):
    mesh = plsc.VectorSubcoreMesh(core_axis_name="c", subcore_axis_name="s",
                                  num_cores=NC, num_subcores=NS)
    run = pl.kernel(
        _sc_body,
        out_type=jax.ShapeDtypeStruct((N,), jnp.float32),
        mesh=mesh,
        compiler_params=pltpu.CompilerParams(needs_layout_passes=False),
        scratch_types=[
            pltpu.VMEM((2 * D,), jnp.float32),
            pltpu.VMEM((CHUNK * D,), jnp.float32),
            pltpu.VMEM((CHUNK * D,), jnp.float32),
            pltpu.VMEM((CHUNK,), jnp.float32),
            pltpu.VMEM((CHUNK,), jnp.float32),
            pltpu.SemaphoreType.DMA,
            pltpu.SemaphoreType.DMA,
            pltpu.SemaphoreType.DMA,
            pltpu.SemaphoreType.DMA,
        ],
    )
    return run(embedded.reshape(N * D), centers.reshape(2 * D))


# speedup vs baseline: 1.4906x; 1.2297x over previous
import jax
import jax.numpy as jnp
from jax import lax
from jax.experimental import pallas as pl
from jax.experimental.pallas import tpu as pltpu
from jax.experimental.pallas import tpu_sc as plsc

N = 131072
D = 32
NC, NS, L = 2, 16, 16
NW = NC * NS
ROWS_PER_W = N // NW
CHUNK = 128
NCHUNK = ROWS_PER_W // CHUNK
NG = 8
SWEEPS = CHUNK // (NG * L)
NBUF = 2


def _sqrt16(x):
    i = lax.bitcast_convert_type(x, jnp.int32)
    y = lax.bitcast_convert_type(jnp.int32(0x5F3759DF) - (i >> 1),
                                 jnp.float32)
    for _ in range(3):
        y = y * (1.5 - 0.5 * x * y * y)
    return x * y


def _sc_body(emb_hbm, cen_hbm, out_hbm,
             cen_v, in_v0, in_v1, out_v0, out_v1,
             isem0, isem1, osem0, osem1):
    wid = lax.axis_index("s") * NC + lax.axis_index("c")
    base = wid * ROWS_PER_W

    in_v = (in_v0, in_v1)
    out_v = (out_v0, out_v1)
    isem = (isem0, isem1)
    osem = (osem0, osem1)

    pltpu.sync_copy(cen_hbm, cen_v)

    iota = lax.iota(jnp.int32, L)
    zero16 = jnp.zeros((L,), jnp.int32)
    one16 = jnp.full((L,), 1, jnp.int32)
    rots = [(iota + j) & (D - 1) for j in range(D)]
    rows = [iota + g * L for g in range(NG * SWEEPS)]

    for b in range(NBUF):
        pltpu.async_copy(emb_hbm.at[pl.ds((base + b * CHUNK), CHUNK)],
                         in_v[b], isem[b])

    @pl.loop(0, NCHUNK, step=NBUF)
    def _chunk_pair(c):
        for b in range(NBUF):
            cid = c + b
            pltpu.make_async_copy(emb_hbm.at[pl.ds(0, CHUNK)],
                                  in_v[b], isem[b]).wait()
            for s in range(SWEEPS):
                acc0 = [jnp.zeros((L,), jnp.float32) for _ in range(NG)]
                acc1 = [jnp.zeros((L,), jnp.float32) for _ in range(NG)]
                for j in range(D):
                    c0 = plsc.load_gather(cen_v, [zero16, rots[j]])
                    c1 = plsc.load_gather(cen_v, [one16, rots[j]])
                    for g in range(NG):
                        x = plsc.load_gather(
                            in_v[b], [rows[s * NG + g], rots[j]])
                        u0 = x - c0
                        acc0[g] = acc0[g] + u0 * u0
                        u1 = x - c1
                        acc1[g] = acc1[g] + u1 * u1
                if s == 0:
                    @pl.when(cid >= NBUF)
                    def _():
                        pltpu.make_async_copy(
                            out_v[b], out_hbm.at[pl.ds(base, CHUNK)],
                            osem[b]).wait()
                for g in range(NG):
                    m = jnp.minimum(acc0[g], acc1[g]) + 1e-12
                    out_v[b][pl.ds((s * NG + g) * L, L)] = _sqrt16(m)
            pltpu.async_copy(out_v[b],
                             out_hbm.at[pl.ds(base + cid * CHUNK, CHUNK)],
                             osem[b])

            @pl.when(cid + NBUF < NCHUNK)
            def _():
                pltpu.async_copy(
                    emb_hbm.at[pl.ds(base + (cid + NBUF) * CHUNK, CHUNK)],
                    in_v[b], isem[b])

    for b in range(NBUF):
        pltpu.make_async_copy(out_v[b], out_hbm.at[pl.ds(base, CHUNK)],
                              osem[b]).wait()


def kernel(embedded, centers):
    mesh = plsc.VectorSubcoreMesh(core_axis_name="c", subcore_axis_name="s",
                                  num_cores=NC, num_subcores=NS)
    run = pl.kernel(
        _sc_body,
        out_type=jax.ShapeDtypeStruct((N,), jnp.float32),
        mesh=mesh,
        compiler_params=pltpu.CompilerParams(needs_layout_passes=False),
        scratch_types=[
            pltpu.VMEM((2, D), jnp.float32),
            pltpu.VMEM((CHUNK, D), jnp.float32),
            pltpu.VMEM((CHUNK, D), jnp.float32),
            pltpu.VMEM((CHUNK,), jnp.float32),
            pltpu.VMEM((CHUNK,), jnp.float32),
            pltpu.SemaphoreType.DMA,
            pltpu.SemaphoreType.DMA,
            pltpu.SemaphoreType.DMA,
            pltpu.SemaphoreType.DMA,
        ],
    )
    return run(embedded, centers)


# speedup vs baseline: 1.4955x; 1.0033x over previous
import jax
import jax.numpy as jnp
from jax import lax
from jax.experimental import pallas as pl
from jax.experimental.pallas import tpu as pltpu
from jax.experimental.pallas import tpu_sc as plsc

N = 131072
D = 32
NC, NS, L = 2, 16, 16
NW = NC * NS
SAMPLES_PER_W = N // NW
SPR = 128 // D
CHUNK_S = 512
CHUNK_R = CHUNK_S // SPR
NCHUNK = SAMPLES_PER_W // CHUNK_S
SWEEP_S = 128
SWEEPS = CHUNK_S // SWEEP_S
NBUF = 4
NOBUF = 2


def _sqrt16(x):
    i = lax.bitcast_convert_type(x, jnp.int32)
    y = lax.bitcast_convert_type(jnp.int32(0x5F3759DF) - (i >> 1),
                                 jnp.float32)
    for _ in range(3):
        y = y * (1.5 - 0.5 * x * y * y)
    return x * y


def _sc_body(emb_hbm, cen_hbm, out_hbm,
             cen_v, in_v0, in_v1, in_v2, in_v3, out_v0, out_v1,
             isem0, isem1, isem2, isem3, osem0, osem1):
    wid = lax.axis_index("s") * NC + lax.axis_index("c")
    rbase = wid * (SAMPLES_PER_W // SPR)
    sbase = wid * SAMPLES_PER_W

    in_v = (in_v0, in_v1, in_v2, in_v3)
    out_v = (out_v0, out_v1)
    isem = (isem0, isem1, isem2, isem3)
    osem = (osem0, osem1)

    pltpu.sync_copy(cen_hbm, cen_v)

    iota = lax.iota(jnp.int32, L)
    zero16 = jnp.zeros((L,), jnp.int32)
    one16 = jnp.full((L,), 1, jnp.int32)
    iota4 = iota * SPR
    rots = [(iota + j) & (D - 1) for j in range(D)]

    for b in range(NBUF):
        pltpu.async_copy(emb_hbm.at[pl.ds(rbase + b * CHUNK_R, CHUNK_R)],
                         in_v[b], isem[b])

    @pl.loop(0, NCHUNK, step=NBUF)
    def _chunk_quad(c):
        for b in range(NBUF):
            cid = c + b
            ob = b & 1
            pltpu.make_async_copy(emb_hbm.at[pl.ds(0, CHUNK_R)],
                                  in_v[b], isem[b]).wait()

            @pl.when(cid >= NOBUF)
            def _():
                pltpu.make_async_copy(out_v[ob],
                                      out_hbm.at[pl.ds(0, CHUNK_S)],
                                      osem[ob]).wait()

            @pl.loop(0, SWEEPS)
            def _sweep(s):
                srow = s * (SWEEP_S // SPR)
                acc0 = [jnp.zeros((L,), jnp.float32) for _ in range(8)]
                acc1 = [jnp.zeros((L,), jnp.float32) for _ in range(8)]
                rows = [iota + (srow + 16 * h) for h in range(2)]
                for j in range(D):
                    c0 = plsc.load_gather(cen_v, [zero16, rots[j]])
                    c1 = plsc.load_gather(cen_v, [one16, rots[j]])
                    cols = [rots[j] + q * D for q in range(SPR)]
                    for g in range(8):
                        h, q = g >> 2, g & 3
                        x = plsc.load_gather(in_v[b], [rows[h], cols[q]])
                        u0 = x - c0
                        acc0[g] = acc0[g] + u0 * u0
                        u1 = x - c1
                        acc1[g] = acc1[g] + u1 * u1
                for g in range(8):
                    h, q = g >> 2, g & 3
                    m = jnp.minimum(acc0[g], acc1[g]) + 1e-12
                    plsc.store_scatter(
                        out_v[ob],
                        [iota4 + (s * SWEEP_S + 64 * h + q)],
                        _sqrt16(m))

            pltpu.async_copy(out_v[ob],
                             out_hbm.at[pl.ds(sbase + cid * CHUNK_S,
                                              CHUNK_S)],
                             osem[ob])

            @pl.when(cid + NBUF < NCHUNK)
            def _():
                pltpu.async_copy(
                    emb_hbm.at[pl.ds(rbase + (cid + NBUF) * CHUNK_R,
                                     CHUNK_R)],
                    in_v[b], isem[b])

    for ob in range(NOBUF):
        pltpu.make_async_copy(out_v[ob], out_hbm.at[pl.ds(0, CHUNK_S)],
                              osem[ob]).wait()


def kernel(embedded, centers):
    mesh = plsc.VectorSubcoreMesh(core_axis_name="c", subcore_axis_name="s",
                                  num_cores=NC, num_subcores=NS)
    run = pl.kernel(
        _sc_body,
        out_type=jax.ShapeDtypeStruct((N,), jnp.float32),
        mesh=mesh,
        compiler_params=pltpu.CompilerParams(needs_layout_passes=False),
        scratch_types=[
            pltpu.VMEM((2, D), jnp.float32),
            pltpu.VMEM((CHUNK_R, 128), jnp.float32),
            pltpu.VMEM((CHUNK_R, 128), jnp.float32),
            pltpu.VMEM((CHUNK_R, 128), jnp.float32),
            pltpu.VMEM((CHUNK_R, 128), jnp.float32),
            pltpu.VMEM((CHUNK_S,), jnp.float32),
            pltpu.VMEM((CHUNK_S,), jnp.float32),
            pltpu.SemaphoreType.DMA,
            pltpu.SemaphoreType.DMA,
            pltpu.SemaphoreType.DMA,
            pltpu.SemaphoreType.DMA,
            pltpu.SemaphoreType.DMA,
            pltpu.SemaphoreType.DMA,
        ],
    )
    return run(embedded.reshape(N // SPR, 128), centers)


# speedup vs baseline: 1.5931x; 1.0652x over previous
import jax
import jax.numpy as jnp
from jax import lax
from jax.experimental import pallas as pl
from jax.experimental.pallas import tpu as pltpu
from jax.experimental.pallas import tpu_sc as plsc

N = 131072
D = 32
NC, NS, L = 2, 16, 16
NW = NC * NS
SAMPLES_PER_W = N // NW
CHUNK_S = 512
NCHUNK = SAMPLES_PER_W // CHUNK_S
SWEEP_S = 128
SWEEPS = CHUNK_S // SWEEP_S
NG = SWEEP_S // L
NBUF = 4
NOBUF = 2


def _sqrt16(x):
    i = lax.bitcast_convert_type(x, jnp.int32)
    y = lax.bitcast_convert_type(jnp.int32(0x5F3759DF) - (i >> 1),
                                 jnp.float32)
    for _ in range(3):
        y = y * (1.5 - 0.5 * x * y * y)
    return x * y


def _sc_body(emb_hbm, cen_hbm, out_hbm,
             cen_v, in_v0, in_v1, in_v2, in_v3, out_v0, out_v1,
             isem0, isem1, isem2, isem3, osem0, osem1):
    wid = lax.axis_index("s") * NC + lax.axis_index("c")
    sbase = wid * SAMPLES_PER_W

    in_v = (in_v0, in_v1, in_v2, in_v3)
    out_v = (out_v0, out_v1)
    isem = (isem0, isem1, isem2, isem3)
    osem = (osem0, osem1)

    pltpu.sync_copy(cen_hbm, cen_v)

    iota = lax.iota(jnp.int32, L)
    zero16 = jnp.zeros((L,), jnp.int32)
    one16 = jnp.full((L,), 1, jnp.int32)
    rots = [(iota + j) & (D - 1) for j in range(D)]

    for b in range(NBUF):
        pltpu.async_copy(emb_hbm.at[pl.ds(sbase + b * CHUNK_S, CHUNK_S)],
                         in_v[b], isem[b])

    @pl.loop(0, NCHUNK, step=NBUF)
    def _chunk_quad(c):
        for b in range(NBUF):
            cid = c + b
            ob = b & 1
            pltpu.make_async_copy(emb_hbm.at[pl.ds(0, CHUNK_S)],
                                  in_v[b], isem[b]).wait()

            @pl.when(cid >= NOBUF)
            def _():
                pltpu.make_async_copy(out_v[ob],
                                      out_hbm.at[pl.ds(0, CHUNK_S)],
                                      osem[ob]).wait()

            @pl.loop(0, SWEEPS)
            def _sweep(s):
                srow = s * SWEEP_S
                acc0 = [jnp.zeros((L,), jnp.float32) for _ in range(NG)]
                acc1 = [jnp.zeros((L,), jnp.float32) for _ in range(NG)]
                rows = [iota + (srow + g * L) for g in range(NG)]
                for j in range(D):
                    c0 = plsc.load_gather(cen_v, [zero16, rots[j]])
                    c1 = plsc.load_gather(cen_v, [one16, rots[j]])
                    for g in range(NG):
                        x = plsc.load_gather(in_v[b], [rows[g], rots[j]])
                        u0 = x - c0
                        acc0[g] = acc0[g] + u0 * u0
                        u1 = x - c1
                        acc1[g] = acc1[g] + u1 * u1
                for g in range(NG):
                    m = jnp.minimum(acc0[g], acc1[g]) + 1e-12
                    out_v[ob][pl.ds(srow + g * L, L)] = _sqrt16(m)

            pltpu.async_copy(out_v[ob],
                             out_hbm.at[pl.ds(sbase + cid * CHUNK_S,
                                              CHUNK_S)],
                             osem[ob])

            @pl.when(cid + NBUF < NCHUNK)
            def _():
                pltpu.async_copy(
                    emb_hbm.at[pl.ds(sbase + (cid + NBUF) * CHUNK_S,
                                     CHUNK_S)],
                    in_v[b], isem[b])

    for ob in range(NOBUF):
        pltpu.make_async_copy(out_v[ob], out_hbm.at[pl.ds(0, CHUNK_S)],
                              osem[ob]).wait()


def kernel(embedded, centers):
    mesh = plsc.VectorSubcoreMesh(core_axis_name="c", subcore_axis_name="s",
                                  num_cores=NC, num_subcores=NS)
    run = pl.kernel(
        _sc_body,
        out_type=jax.ShapeDtypeStruct((N,), jnp.float32),
        mesh=mesh,
        compiler_params=pltpu.CompilerParams(needs_layout_passes=False,
                                             use_tc_tiling_on_sc=False),
        scratch_types=[
            pltpu.VMEM((2, D), jnp.float32),
            pltpu.VMEM((CHUNK_S, D), jnp.float32),
            pltpu.VMEM((CHUNK_S, D), jnp.float32),
            pltpu.VMEM((CHUNK_S, D), jnp.float32),
            pltpu.VMEM((CHUNK_S, D), jnp.float32),
            pltpu.VMEM((CHUNK_S,), jnp.float32),
            pltpu.VMEM((CHUNK_S,), jnp.float32),
            pltpu.SemaphoreType.DMA,
            pltpu.SemaphoreType.DMA,
            pltpu.SemaphoreType.DMA,
            pltpu.SemaphoreType.DMA,
            pltpu.SemaphoreType.DMA,
            pltpu.SemaphoreType.DMA,
        ],
    )
    return run(embedded, centers)


# speedup vs baseline: 1.6077x; 1.0092x over previous
import jax
import jax.numpy as jnp
from jax import lax
from jax.experimental import pallas as pl
from jax.experimental.pallas import tpu as pltpu
from jax.experimental.pallas import tpu_sc as plsc

N = 131072
D = 32
NC, NS, L = 2, 16, 16
NW = NC * NS
SAMPLES_PER_W = N // NW
CHUNK_S = 512
NCHUNK = SAMPLES_PER_W // CHUNK_S
SWEEP_S = 128
SWEEPS = CHUNK_S // SWEEP_S
NG = SWEEP_S // L
NBUF = 4
NOBUF = 2


def _sqrt16(x):
    i = lax.bitcast_convert_type(x, jnp.int32)
    y = lax.bitcast_convert_type(jnp.int32(0x5F3759DF) - (i >> 1),
                                 jnp.float32)
    for _ in range(3):
        y = y * (1.5 - 0.5 * x * y * y)
    return x * y


def _sc_body(emb_hbm, cen_hbm, out_hbm,
             cen_v, in_v0, in_v1, in_v2, in_v3, out_v0, out_v1,
             isem0, isem1, isem2, isem3, osem0, osem1):
    wid = lax.axis_index("s") * NC + lax.axis_index("c")
    sbase = wid * SAMPLES_PER_W

    in_v = (in_v0, in_v1, in_v2, in_v3)
    out_v = (out_v0, out_v1)
    isem = (isem0, isem1, isem2, isem3)
    osem = (osem0, osem1)

    pltpu.sync_copy(cen_hbm, cen_v)

    iota = lax.iota(jnp.int32, L)
    zero16 = jnp.zeros((L,), jnp.int32)
    one16 = jnp.full((L,), 1, jnp.int32)
    rots = [(iota + j) & (D - 1) for j in range(D)]

    for b in range(NBUF):
        pltpu.async_copy(emb_hbm.at[pl.ds(sbase + b * CHUNK_S, CHUNK_S)],
                         in_v[b], isem[b])

    @pl.loop(0, NCHUNK, step=NBUF)
    def _chunk_quad(c):
        for b in range(NBUF):
            cid = c + b
            ob = b & 1
            pltpu.make_async_copy(emb_hbm.at[pl.ds(0, CHUNK_S)],
                                  in_v[b], isem[b]).wait()

            @pl.when(cid >= NOBUF)
            def _():
                pltpu.make_async_copy(out_v[ob],
                                      out_hbm.at[pl.ds(0, CHUNK_S)],
                                      osem[ob]).wait()

            @pl.loop(0, SWEEPS)
            def _sweep(s):
                srow = s * SWEEP_S
                zeros = tuple(jnp.zeros((L,), jnp.float32)
                              for _ in range(2 * NG))
                rows = [iota + (srow + g * L) for g in range(NG)]

                @pl.loop(0, D, init_carry=zeros, unroll=4)
                def _feat(j, carry):
                    rot = (iota + j) & (D - 1)
                    c0 = plsc.load_gather(cen_v, [zero16, rot])
                    c1 = plsc.load_gather(cen_v, [one16, rot])
                    acc = list(carry)
                    for g in range(NG):
                        x = plsc.load_gather(in_v[b], [rows[g], rot])
                        u0 = x - c0
                        acc[2 * g] = acc[2 * g] + u0 * u0
                        u1 = x - c1
                        acc[2 * g + 1] = acc[2 * g + 1] + u1 * u1
                    return tuple(acc)

                for g in range(NG):
                    m = jnp.minimum(_feat[2 * g], _feat[2 * g + 1]) + 1e-12
                    out_v[ob][pl.ds(srow + g * L, L)] = _sqrt16(m)

            pltpu.async_copy(out_v[ob],
                             out_hbm.at[pl.ds(sbase + cid * CHUNK_S,
                                              CHUNK_S)],
                             osem[ob])

            @pl.when(cid + NBUF < NCHUNK)
            def _():
                pltpu.async_copy(
                    emb_hbm.at[pl.ds(sbase + (cid + NBUF) * CHUNK_S,
                                     CHUNK_S)],
                    in_v[b], isem[b])

    for ob in range(NOBUF):
        pltpu.make_async_copy(out_v[ob], out_hbm.at[pl.ds(0, CHUNK_S)],
                              osem[ob]).wait()


def kernel(embedded, centers):
    mesh = plsc.VectorSubcoreMesh(core_axis_name="c", subcore_axis_name="s",
                                  num_cores=NC, num_subcores=NS)
    run = pl.kernel(
        _sc_body,
        out_type=jax.ShapeDtypeStruct((N,), jnp.float32),
        mesh=mesh,
        compiler_params=pltpu.CompilerParams(needs_layout_passes=False,
                                             use_tc_tiling_on_sc=False),
        scratch_types=[
            pltpu.VMEM((2, D), jnp.float32),
            pltpu.VMEM((CHUNK_S, D), jnp.float32),
            pltpu.VMEM((CHUNK_S, D), jnp.float32),
            pltpu.VMEM((CHUNK_S, D), jnp.float32),
            pltpu.VMEM((CHUNK_S, D), jnp.float32),
            pltpu.VMEM((CHUNK_S,), jnp.float32),
            pltpu.VMEM((CHUNK_S,), jnp.float32),
            pltpu.SemaphoreType.DMA,
            pltpu.SemaphoreType.DMA,
            pltpu.SemaphoreType.DMA,
            pltpu.SemaphoreType.DMA,
            pltpu.SemaphoreType.DMA,
            pltpu.SemaphoreType.DMA,
        ],
    )
    return run(embedded, centers)
